# Initial kernel scaffold; baseline (speedup 1.0000x reference)
import jax
import jax.numpy as jnp
from jax.experimental import pallas as pl


def kernel(x, edge_index, batch, W1, b1, g1, be1, W2, b2, g2, be2, W3, b3, g3, be3, Wl, bl):
    raise NotImplementedError("write your pallas kernel here")


# speedup vs baseline: 10.2267x; 10.2267x over previous
import functools
import math

import jax
import jax.numpy as jnp
from jax import lax
from jax.experimental import pallas as pl
from jax.experimental.pallas import tpu as pltpu
from jax.experimental.pallas import tpu_sc as plsc

N = 10000
D = 128
NG = 64
NC, NS = 2, 16
NW = NC * NS
CHUNK = 128
N_PAD = 10112
ROWS_T = N_PAD // NS
EPS = 1e-5
BR = 400
GRID = N // BR


def _fill_const(ref, rows, cols, val):
    vec = jnp.full((16,), val, jnp.float32)

    def row(r, _):
        def col(c, __):
            ref[r, pl.ds(c * 16, 16)] = vec
            return 0
        return lax.fori_loop(0, cols // 16, col, 0)
    lax.fori_loop(0, rows, row, 0)


def _sc_deg_body(dst_hbm, out_hbm, dst_v, ones_v, zero_v, acc_s, sem):
    c = lax.axis_index("c")
    s = lax.axis_index("s")
    jcount = dst_hbm.shape[2]

    pltpu.async_copy(dst_hbm.at[c, s], dst_v, sem).wait()
    _fill_const(ones_v, CHUNK, 16, 1.0)
    _fill_const(zero_v, CHUNK, 16, 0.0)

    base = s * ROWS_T
    nfull = ROWS_T // CHUNK
    rem = ROWS_T - nfull * CHUNK

    def zb(k, _):
        pltpu.sync_copy(zero_v, acc_s.at[pl.ds(base + k * CHUNK, CHUNK)])
        return 0
    lax.fori_loop(0, nfull, zb, 0)
    pltpu.sync_copy(zero_v.at[pl.ds(0, rem)], acc_s.at[pl.ds(base + nfull * CHUNK, rem)])

    plsc.subcore_barrier()

    def body(j, _):
        pltpu.sync_copy(ones_v, acc_s.at[dst_v.at[j]], add=True)
        return 0
    lax.fori_loop(0, jcount, body, 0)

    plsc.subcore_barrier()
    pltpu.sync_copy(acc_s.at[pl.ds(base, ROWS_T)], out_hbm.at[c, pl.ds(base, ROWS_T)])


def _sc_deg(dst_r):
    jcount = dst_r.shape[2]
    mesh = plsc.VectorSubcoreMesh(core_axis_name="c", subcore_axis_name="s")
    return pl.kernel(
        _sc_deg_body,
        out_type=jax.ShapeDtypeStruct((NC, N_PAD, 16), jnp.float32),
        mesh=mesh,
        scratch_types=[
            pltpu.VMEM((jcount, CHUNK), jnp.int32),
            pltpu.VMEM((CHUNK, 16), jnp.float32),
            pltpu.VMEM((CHUNK, 16), jnp.float32),
            pltpu.VMEM_SHARED((N_PAD, 16), jnp.float32),
            pltpu.SemaphoreType.DMA,
        ],
    )(dst_r)


def _sc_agg_body(y_hbm, src_hbm, dst_hbm, out_hbm, src_v, dst_v, rows_v, acc_s, sem):
    c = lax.axis_index("c")
    s = lax.axis_index("s")
    jcount = src_hbm.shape[2]

    pltpu.async_copy(src_hbm.at[c, s], src_v, sem).wait()
    pltpu.async_copy(dst_hbm.at[c, s], dst_v, sem).wait()

    _fill_const(rows_v, CHUNK, D, 0.0)
    base = s * ROWS_T
    nfull = ROWS_T // CHUNK
    rem = ROWS_T - nfull * CHUNK

    def zb(k, _):
        pltpu.sync_copy(rows_v, acc_s.at[pl.ds(base + k * CHUNK, CHUNK)])
        return 0
    lax.fori_loop(0, nfull, zb, 0)
    pltpu.sync_copy(rows_v.at[pl.ds(0, rem)], acc_s.at[pl.ds(base + nfull * CHUNK, rem)])

    plsc.subcore_barrier()

    def body(j, _):
        pltpu.async_copy(y_hbm.at[src_v.at[j]], rows_v, sem).wait()
        pltpu.sync_copy(rows_v, acc_s.at[dst_v.at[j]], add=True)
        return 0
    lax.fori_loop(0, jcount, body, 0)

    plsc.subcore_barrier()
    pltpu.sync_copy(acc_s.at[pl.ds(base, ROWS_T)], out_hbm.at[c, pl.ds(base, ROWS_T)])


def _sc_agg(y, src_r, dst_r):
    jcount = src_r.shape[2]
    mesh = plsc.VectorSubcoreMesh(core_axis_name="c", subcore_axis_name="s")
    return pl.kernel(
        _sc_agg_body,
        out_type=jax.ShapeDtypeStruct((NC, N_PAD, D), jnp.float32),
        mesh=mesh,
        scratch_types=[
            pltpu.VMEM((jcount, CHUNK), jnp.int32),
            pltpu.VMEM((jcount, CHUNK), jnp.int32),
            pltpu.VMEM((CHUNK, D), jnp.float32),
            pltpu.VMEM_SHARED((N_PAD, D), jnp.float32),
            pltpu.SemaphoreType.DMA,
        ],
    )(y, src_r, dst_r)


def _tc1_body(degp_ref, x_ref, w_ref, y_ref, dinv_ref):
    deg = degp_ref[0, :, 0:1] + degp_ref[1, :, 0:1] + 1.0
    dinv = lax.rsqrt(deg)
    xw = jnp.dot(x_ref[...], w_ref[...], preferred_element_type=jnp.float32)
    y_ref[...] = dinv * xw
    dinv_ref[...] = dinv


def _tc1(degp, x, w):
    return pl.pallas_call(
        _tc1_body,
        grid=(GRID,),
        in_specs=[
            pl.BlockSpec((NC, BR, 16), lambda i: (0, i, 0)),
            pl.BlockSpec((BR, D), lambda i: (i, 0)),
            pl.BlockSpec((D, D), lambda i: (0, 0)),
        ],
        out_specs=[
            pl.BlockSpec((BR, D), lambda i: (i, 0)),
            pl.BlockSpec((BR, 1), lambda i: (i, 0)),
        ],
        out_shape=[
            jax.ShapeDtypeStruct((N, D), jnp.float32),
            jax.ShapeDtypeStruct((N, 1), jnp.float32),
        ],
    )(degp, x, w)


def _ln_relu(s, g, be):
    mu = jnp.mean(s, axis=1, keepdims=True)
    cen = s - mu
    var = jnp.mean(cen * cen, axis=1, keepdims=True)
    return jnp.maximum(cen * lax.rsqrt(var + EPS) * g + be, 0.0)


def _tc_mid_body(p_ref, yp_ref, dinv_ref, b_ref, g_ref, be_ref, w_ref, out_ref):
    dinv = dinv_ref[...]
    s = dinv * (p_ref[0] + p_ref[1] + yp_ref[...]) + b_ref[...]
    h = _ln_relu(s, g_ref[...], be_ref[...])
    out_ref[...] = dinv * jnp.dot(h, w_ref[...], preferred_element_type=jnp.float32)


def _tc_mid(p, yp, dinv, b, g, be, w):
    return pl.pallas_call(
        _tc_mid_body,
        grid=(GRID,),
        in_specs=[
            pl.BlockSpec((NC, BR, D), lambda i: (0, i, 0)),
            pl.BlockSpec((BR, D), lambda i: (i, 0)),
            pl.BlockSpec((BR, 1), lambda i: (i, 0)),
            pl.BlockSpec((1, D), lambda i: (0, 0)),
            pl.BlockSpec((1, D), lambda i: (0, 0)),
            pl.BlockSpec((1, D), lambda i: (0, 0)),
            pl.BlockSpec((D, D), lambda i: (0, 0)),
        ],
        out_specs=pl.BlockSpec((BR, D), lambda i: (i, 0)),
        out_shape=jax.ShapeDtypeStruct((N, D), jnp.float32),
    )(p, yp, dinv, b, g, be, w)


def _tc_final_body(p_ref, yp_ref, dinv_ref, b_ref, g_ref, be_ref, batch_ref,
                   wl_ref, bl_ref, out_ref, accs_ref, accc_ref):
    i = pl.program_id(0)

    @pl.when(i == 0)
    def _():
        accs_ref[...] = jnp.zeros((NG, D), jnp.float32)
        accc_ref[...] = jnp.zeros((NG, 1), jnp.float32)

    dinv = dinv_ref[...]
    s = dinv * (p_ref[0] + p_ref[1] + yp_ref[...]) + b_ref[...]
    h = _ln_relu(s, g_ref[...], be_ref[...])

    onehot = (lax.broadcasted_iota(jnp.int32, (NG, BR), 0)
              == batch_ref[0]).astype(jnp.float32)
    accs_ref[...] += jnp.dot(onehot, h, preferred_element_type=jnp.float32)
    accc_ref[...] += jnp.sum(onehot, axis=1, keepdims=True)

    @pl.when(i == GRID - 1)
    def _():
        pooled = accs_ref[...] / jnp.maximum(accc_ref[...], 1.0)
        out_ref[...] = (jnp.dot(pooled, wl_ref[...],
                                preferred_element_type=jnp.float32) + bl_ref[...])


def _tc_final(p, yp, dinv, b, g, be, batch2d, wl, bl2d):
    return pl.pallas_call(
        _tc_final_body,
        grid=(GRID,),
        in_specs=[
            pl.BlockSpec((NC, BR, D), lambda i: (0, i, 0)),
            pl.BlockSpec((BR, D), lambda i: (i, 0)),
            pl.BlockSpec((BR, 1), lambda i: (i, 0)),
            pl.BlockSpec((1, D), lambda i: (0, 0)),
            pl.BlockSpec((1, D), lambda i: (0, 0)),
            pl.BlockSpec((1, D), lambda i: (0, 0)),
            pl.BlockSpec((1, 1, BR), lambda i: (i, 0, 0)),
            pl.BlockSpec((D, 1), lambda i: (0, 0)),
            pl.BlockSpec((1, 1), lambda i: (0, 0)),
        ],
        out_specs=pl.BlockSpec((NG, 1), lambda i: (0, 0)),
        out_shape=jax.ShapeDtypeStruct((NG, 1), jnp.float32),
        scratch_shapes=[
            pltpu.VMEM((NG, D), jnp.float32),
            pltpu.VMEM((NG, 1), jnp.float32),
        ],
    )(p, yp, dinv, b, g, be, batch2d, wl, bl2d)


def kernel(x, edge_index, batch, W1, b1, g1, be1, W2, b2, g2, be2,
           W3, b3, g3, be3, Wl, bl):
    src = edge_index[0].astype(jnp.int32)
    dst = edge_index[1].astype(jnp.int32)
    e = src.shape[0]
    jcount = math.ceil(e / (NW * CHUNK))
    e_pad = NW * jcount * CHUNK
    src_p = jnp.concatenate([src, jnp.zeros((e_pad - e,), jnp.int32)])
    dst_p = jnp.concatenate([dst, jnp.full((e_pad - e,), N, jnp.int32)])
    src_r = src_p.reshape(NC, NS, jcount, CHUNK)
    dst_r = dst_p.reshape(NC, NS, jcount, CHUNK)

    batch2d = batch.astype(jnp.int32).reshape(GRID, 1, BR)
    b1r, g1r, be1r = b1.reshape(1, D), g1.reshape(1, D), be1.reshape(1, D)
    b2r, g2r, be2r = b2.reshape(1, D), g2.reshape(1, D), be2.reshape(1, D)
    b3r, g3r, be3r = b3.reshape(1, D), g3.reshape(1, D), be3.reshape(1, D)
    bl2d = bl.reshape(1, 1)

    degp = _sc_deg(dst_r)
    y1, dinv = _tc1(degp, x, W1)
    p1 = _sc_agg(y1, src_r, dst_r)
    y2 = _tc_mid(p1, y1, dinv, b1r, g1r, be1r, W2)
    p2 = _sc_agg(y2, src_r, dst_r)
    y3 = _tc_mid(p2, y2, dinv, b2r, g2r, be2r, W3)
    p3 = _sc_agg(y3, src_r, dst_r)
    return _tc_final(p3, y3, dinv, b3r, g3r, be3r, batch2d, Wl, bl2d)
